# Initial kernel scaffold; baseline (speedup 1.0000x reference)
import jax
import jax.numpy as jnp
from jax.experimental import pallas as pl


def kernel(x, edge_index, edge_values, W, b):
    raise NotImplementedError("write your pallas kernel here")


# speedup vs baseline: 3.2779x; 3.2779x over previous
import jax
import jax.numpy as jnp
from jax import lax
from jax.experimental import pallas as pl
from jax.experimental.pallas import tpu as pltpu
from jax.experimental.pallas import tpu_sc as plsc

N = 10000
E = 160000
D = 256
HALF = 128
NTILES = 16
K = 128
NB = 80
NCH = 8
CH = NB // NCH
E_PAD = NTILES * NB * K
ROWS_PT = 624
REM_BASE = NTILES * ROWS_PT
REM = N - REM_BASE


def _linear_kernel(x_ref, w_ref, b_ref, o0_ref, o1_ref):
    y = lax.dot_general(x_ref[...], w_ref[...], (((1,), (1,)), ((), ())),
                        preferred_element_type=jnp.float32)
    y = y + b_ref[...]
    o0_ref[...] = y[:, :HALF]
    o1_ref[...] = y[:, HALF:]


def _linear(x, W, b):
    RB = 1000
    return pl.pallas_call(
        _linear_kernel,
        grid=(N // RB,),
        in_specs=[
            pl.BlockSpec((RB, D), lambda i: (i, 0)),
            pl.BlockSpec((D, D), lambda i: (0, 0)),
            pl.BlockSpec((1, D), lambda i: (0, 0)),
        ],
        out_specs=[
            pl.BlockSpec((RB, HALF), lambda i: (i, 0)),
            pl.BlockSpec((RB, HALF), lambda i: (i, 0)),
        ],
        out_shape=[
            jax.ShapeDtypeStruct((N, HALF), jnp.float32),
            jax.ShapeDtypeStruct((N, HALF), jnp.float32),
        ],
    )(x, W, b.reshape(1, D))


def _sc_body(xt0, xt1, col_r, row_r, ev_r, out,
             acc, colv, rowv, evv, buf0, buf1, sem0, sem1):
    c = lax.axis_index("c")
    s = lax.axis_index("s")

    zero16 = jnp.zeros((16,), jnp.float32)

    def _zrow(i, carry):
        for v in range(HALF // 16):
            buf0[i, pl.ds(v * 16, 16)] = zero16
        return carry

    lax.fori_loop(0, K, _zrow, 0)
    nfull = ROWS_PT // K
    for k in range(nfull):
        pltpu.sync_copy(buf0, acc.at[pl.ds(s * ROWS_PT + k * K, K)])
    rem = ROWS_PT - nfull * K
    if rem:
        pltpu.sync_copy(buf0.at[pl.ds(0, rem)],
                        acc.at[pl.ds(s * ROWS_PT + nfull * K, rem)])

    @pl.when(s == 0)
    def _():
        pltpu.sync_copy(buf0.at[pl.ds(0, REM)], acc.at[pl.ds(REM_BASE, REM)])

    plsc.subcore_barrier()

    bufs = (buf0, buf1)
    sems = (sem0, sem1)

    def _run(xt, col0):
        def gather_start(j, b):
            pltpu.make_async_copy(xt.at[colv.at[j]], bufs[b], sems[b]).start()

        def chunk_body(ch, carry):
            pltpu.sync_copy(col_r.at[s].at[ch], colv)
            pltpu.sync_copy(row_r.at[s].at[ch], rowv)
            pltpu.sync_copy(ev_r.at[s].at[ch], evv)
            gather_start(0, 0)

            def pair_body(t, cc):
                for ph in range(2):
                    j = t * 2 + ph
                    buf = bufs[ph]
                    sem = sems[ph]

                    @pl.when(j + 1 < CH)
                    def _():
                        gather_start(j + 1, 1 - ph)

                    pltpu.make_async_copy(xt.at[colv.at[j]], buf, sem).wait()

                    def scale(g, u):
                        ev16 = evv[j, pl.ds(g * 16, 16)]
                        for i in range(16):
                            val = ev16[i]
                            e = g * 16 + i
                            for v in range(HALF // 16):
                                sl = buf[e, pl.ds(v * 16, 16)]
                                buf[e, pl.ds(v * 16, 16)] = sl * val
                        return u

                    lax.fori_loop(0, K // 16, scale, 0)
                    pltpu.sync_copy(buf, acc.at[rowv.at[j]], add=True)
                return cc

            lax.fori_loop(0, CH // 2, pair_body, 0)
            return carry

        lax.fori_loop(0, NCH, chunk_body, 0)
        plsc.subcore_barrier()
        pltpu.sync_copy(acc.at[pl.ds(s * ROWS_PT, ROWS_PT)],
                        out.at[pl.ds(s * ROWS_PT, ROWS_PT), pl.ds(col0, HALF)])

        @pl.when(s == 0)
        def _():
            pltpu.sync_copy(acc.at[pl.ds(REM_BASE, REM)],
                            out.at[pl.ds(REM_BASE, REM), pl.ds(col0, HALF)])

    @pl.when(c == 0)
    def _():
        _run(xt0, 0)

    @pl.when(c == 1)
    def _():
        _run(xt1, HALF)


def _aggregate(xt0, xt1, col4, row4, ev4):
    mesh = plsc.VectorSubcoreMesh(core_axis_name="c", subcore_axis_name="s")
    return pl.kernel(
        _sc_body,
        out_type=jax.ShapeDtypeStruct((N, D), jnp.float32),
        mesh=mesh,
        scratch_types=[
            pltpu.VMEM_SHARED((N, HALF), jnp.float32),
            pltpu.VMEM((CH, K), jnp.int32),
            pltpu.VMEM((CH, K), jnp.int32),
            pltpu.VMEM((CH, K), jnp.float32),
            pltpu.VMEM((K, HALF), jnp.float32),
            pltpu.VMEM((K, HALF), jnp.float32),
            pltpu.SemaphoreType.DMA,
            pltpu.SemaphoreType.DMA,
        ],
    )(xt0, xt1, col4, row4, ev4)


def kernel(x, edge_index, edge_values, W, b):
    xt0, xt1 = _linear(x, W, b)
    pad = E_PAD - E
    col = jnp.concatenate(
        [edge_index[1].astype(jnp.int32), jnp.zeros((pad,), jnp.int32)])
    row = jnp.concatenate(
        [edge_index[0].astype(jnp.int32), jnp.zeros((pad,), jnp.int32)])
    ev = jnp.concatenate([edge_values, jnp.zeros((pad,), jnp.float32)])
    col4 = col.reshape(NTILES, NCH, CH, K)
    row4 = row.reshape(NTILES, NCH, CH, K)
    ev4 = ev.reshape(NTILES, NCH, CH, K)
    return _aggregate(xt0, xt1, col4, row4, ev4)


# speedup vs baseline: 3.3149x; 1.0113x over previous
import jax
import jax.numpy as jnp
from jax import lax
from jax.experimental import pallas as pl
from jax.experimental.pallas import tpu as pltpu
from jax.experimental.pallas import tpu_sc as plsc

N = 10000
E = 160000
D = 256
HALF = 128
NTILES = 16
K = 64
NB = 160
NCH = 10
CH = NB // NCH
E_PAD = NTILES * NB * K
ROWS_PT = 624
REM_BASE = NTILES * ROWS_PT
REM = N - REM_BASE


def _linear_kernel(x_ref, w_ref, b_ref, o0_ref, o1_ref):
    y = lax.dot_general(x_ref[...], w_ref[...], (((1,), (1,)), ((), ())),
                        preferred_element_type=jnp.float32)
    y = y + b_ref[...]
    o0_ref[...] = y[:, :HALF]
    o1_ref[...] = y[:, HALF:]


def _linear(x, W, b):
    RB = 1000
    return pl.pallas_call(
        _linear_kernel,
        grid=(N // RB,),
        in_specs=[
            pl.BlockSpec((RB, D), lambda i: (i, 0)),
            pl.BlockSpec((D, D), lambda i: (0, 0)),
            pl.BlockSpec((1, D), lambda i: (0, 0)),
        ],
        out_specs=[
            pl.BlockSpec((RB, HALF), lambda i: (i, 0)),
            pl.BlockSpec((RB, HALF), lambda i: (i, 0)),
        ],
        out_shape=[
            jax.ShapeDtypeStruct((N, HALF), jnp.float32),
            jax.ShapeDtypeStruct((N, HALF), jnp.float32),
        ],
    )(x, W, b.reshape(1, D))


def _sc_body(xt0, xt1, col_r, row_r, ev_r, out,
             acc, colv, rowv, evv, gbuf0, gbuf1, sbuf0, sbuf1,
             gsem0, gsem1, ssem0, ssem1):
    c = lax.axis_index("c")
    s = lax.axis_index("s")

    zero16 = jnp.zeros((16,), jnp.float32)

    def _zrow(i, carry):
        for v in range(HALF // 16):
            sbuf0[i, pl.ds(v * 16, 16)] = zero16
        return carry

    lax.fori_loop(0, K, _zrow, 0)
    nfull = ROWS_PT // K
    for k in range(nfull):
        pltpu.sync_copy(sbuf0, acc.at[pl.ds(s * ROWS_PT + k * K, K)])
    rem = ROWS_PT - nfull * K
    if rem:
        pltpu.sync_copy(sbuf0.at[pl.ds(0, rem)],
                        acc.at[pl.ds(s * ROWS_PT + nfull * K, rem)])

    @pl.when(s == 0)
    def _():
        pltpu.sync_copy(sbuf0.at[pl.ds(0, REM)], acc.at[pl.ds(REM_BASE, REM)])

    plsc.subcore_barrier()

    gbufs = (gbuf0, gbuf1)
    gsems = (gsem0, gsem1)
    sbufs = (sbuf0, sbuf1)
    ssems = (ssem0, ssem1)

    def _run(xt, col0):
        def gather_start(j, b):
            pltpu.make_async_copy(xt.at[colv.at[j]], gbufs[b], gsems[b]).start()

        def chunk_body(ch, carry):
            pltpu.sync_copy(col_r.at[s].at[ch], colv)
            pltpu.sync_copy(row_r.at[s].at[ch], rowv)
            pltpu.sync_copy(ev_r.at[s].at[ch], evv)
            gather_start(0, 0)

            def pair_body(t, cc):
                for ph in range(2):
                    j = t * 2 + ph
                    gbuf = gbufs[ph]
                    sbuf = sbufs[ph]

                    @pl.when(j + 1 < CH)
                    def _():
                        gather_start(j + 1, 1 - ph)

                    @pl.when(t > 0)
                    def _():
                        pltpu.make_async_copy(
                            sbuf, acc.at[rowv.at[j]], ssems[ph]).wait()

                    pltpu.make_async_copy(xt.at[colv.at[j]], gbuf,
                                          gsems[ph]).wait()

                    def scale(g, u):
                        ev16 = evv[j, pl.ds(g * 16, 16)]
                        for i in range(16):
                            val = ev16[i]
                            e = g * 16 + i
                            for v in range(HALF // 16):
                                sl = gbuf[e, pl.ds(v * 16, 16)]
                                sbuf[e, pl.ds(v * 16, 16)] = sl * val
                        return u

                    lax.fori_loop(0, K // 16, scale, 0)
                    pltpu.async_copy(sbuf, acc.at[rowv.at[j]], ssems[ph],
                                     add=True)
                return cc

            lax.fori_loop(0, CH // 2, pair_body, 0)
            for ph in range(2):
                pltpu.make_async_copy(sbufs[ph], acc.at[rowv.at[CH - 2 + ph]],
                                      ssems[ph]).wait()
            return carry

        lax.fori_loop(0, NCH, chunk_body, 0)
        plsc.subcore_barrier()
        pltpu.sync_copy(acc.at[pl.ds(s * ROWS_PT, ROWS_PT)],
                        out.at[pl.ds(s * ROWS_PT, ROWS_PT), pl.ds(col0, HALF)])

        @pl.when(s == 0)
        def _():
            pltpu.sync_copy(acc.at[pl.ds(REM_BASE, REM)],
                            out.at[pl.ds(REM_BASE, REM), pl.ds(col0, HALF)])

    @pl.when(c == 0)
    def _():
        _run(xt0, 0)

    @pl.when(c == 1)
    def _():
        _run(xt1, HALF)


def _aggregate(xt0, xt1, col4, row4, ev4):
    mesh = plsc.VectorSubcoreMesh(core_axis_name="c", subcore_axis_name="s")
    return pl.kernel(
        _sc_body,
        out_type=jax.ShapeDtypeStruct((N, D), jnp.float32),
        mesh=mesh,
        scratch_types=[
            pltpu.VMEM_SHARED((N, HALF), jnp.float32),
            pltpu.VMEM((CH, K), jnp.int32),
            pltpu.VMEM((CH, K), jnp.int32),
            pltpu.VMEM((CH, K), jnp.float32),
            pltpu.VMEM((K, HALF), jnp.float32),
            pltpu.VMEM((K, HALF), jnp.float32),
            pltpu.VMEM((K, HALF), jnp.float32),
            pltpu.VMEM((K, HALF), jnp.float32),
            pltpu.SemaphoreType.DMA,
            pltpu.SemaphoreType.DMA,
            pltpu.SemaphoreType.DMA,
            pltpu.SemaphoreType.DMA,
        ],
    )(xt0, xt1, col4, row4, ev4)


def kernel(x, edge_index, edge_values, W, b):
    xt0, xt1 = _linear(x, W, b)
    pad = E_PAD - E
    col = jnp.concatenate(
        [edge_index[1].astype(jnp.int32), jnp.zeros((pad,), jnp.int32)])
    row = jnp.concatenate(
        [edge_index[0].astype(jnp.int32), jnp.zeros((pad,), jnp.int32)])
    ev = jnp.concatenate([edge_values, jnp.zeros((pad,), jnp.float32)])
    col4 = col.reshape(NTILES, NCH, CH, K)
    row4 = row.reshape(NTILES, NCH, CH, K)
    ev4 = ev.reshape(NTILES, NCH, CH, K)
    return _aggregate(xt0, xt1, col4, row4, ev4)


# speedup vs baseline: 3.4470x; 1.0398x over previous
import jax
import jax.numpy as jnp
from jax import lax
from jax.experimental import pallas as pl
from jax.experimental.pallas import tpu as pltpu
from jax.experimental.pallas import tpu_sc as plsc

N = 10000
E = 160000
D = 256
HALF = 128
NTILES = 16
K = 64
NB = 160
NCH = 5
CH = NB // NCH
E_PAD = NTILES * NB * K
ROWS_PT = 624
REM_BASE = NTILES * ROWS_PT
REM = N - REM_BASE


def _linear_kernel(x_ref, w_ref, b_ref, o0_ref, o1_ref):
    y = lax.dot_general(x_ref[...], w_ref[...], (((1,), (1,)), ((), ())),
                        preferred_element_type=jnp.float32)
    y = y + b_ref[...]
    o0_ref[...] = y[:, :HALF]
    o1_ref[...] = y[:, HALF:]


def _linear(x, W, b):
    RB = 1000
    return pl.pallas_call(
        _linear_kernel,
        grid=(N // RB,),
        in_specs=[
            pl.BlockSpec((RB, D), lambda i: (i, 0)),
            pl.BlockSpec((D, D), lambda i: (0, 0)),
            pl.BlockSpec((1, D), lambda i: (0, 0)),
        ],
        out_specs=[
            pl.BlockSpec((RB, HALF), lambda i: (i, 0)),
            pl.BlockSpec((RB, HALF), lambda i: (i, 0)),
        ],
        out_shape=[
            jax.ShapeDtypeStruct((N, HALF), jnp.float32),
            jax.ShapeDtypeStruct((N, HALF), jnp.float32),
        ],
    )(x, W, b.reshape(1, D))


def _sc_body(xt0, xt1, col_r, row_r, ev_r, out,
             acc, colv, rowv, evv, gbuf0, gbuf1, sbuf0, sbuf1,
             gsem0, gsem1, ssem0, ssem1):
    c = lax.axis_index("c")
    s = lax.axis_index("s")

    zero16 = jnp.zeros((16,), jnp.float32)

    def _zrow(i, carry):
        for v in range(HALF // 16):
            sbuf0[i, pl.ds(v * 16, 16)] = zero16
        return carry

    lax.fori_loop(0, K, _zrow, 0)
    nfull = ROWS_PT // K
    for k in range(nfull):
        pltpu.sync_copy(sbuf0, acc.at[pl.ds(s * ROWS_PT + k * K, K)])
    rem = ROWS_PT - nfull * K
    if rem:
        pltpu.sync_copy(sbuf0.at[pl.ds(0, rem)],
                        acc.at[pl.ds(s * ROWS_PT + nfull * K, rem)])

    @pl.when(s == 0)
    def _():
        pltpu.sync_copy(sbuf0.at[pl.ds(0, REM)], acc.at[pl.ds(REM_BASE, REM)])

    plsc.subcore_barrier()

    gbufs = (gbuf0, gbuf1)
    gsems = (gsem0, gsem1)
    sbufs = (sbuf0, sbuf1)
    ssems = (ssem0, ssem1)

    def _run(xt, col0):
        def gather_start(j, b):
            pltpu.make_async_copy(xt.at[colv.at[j]], gbufs[b], gsems[b]).start()

        def chunk_body(ch, carry):
            pltpu.sync_copy(col_r.at[s].at[ch], colv)
            pltpu.sync_copy(row_r.at[s].at[ch], rowv)
            pltpu.sync_copy(ev_r.at[s].at[ch], evv)
            gather_start(0, 0)

            def pair_body(t, cc):
                for ph in range(2):
                    j = t * 2 + ph
                    gbuf = gbufs[ph]
                    sbuf = sbufs[ph]

                    @pl.when(j + 1 < CH)
                    def _():
                        gather_start(j + 1, 1 - ph)

                    @pl.when(t > 0)
                    def _():
                        pltpu.make_async_copy(
                            sbuf, acc.at[rowv.at[j]], ssems[ph]).wait()

                    pltpu.make_async_copy(xt.at[colv.at[j]], gbuf,
                                          gsems[ph]).wait()

                    def scale(g, u):
                        ev16 = evv[j, pl.ds(g * 16, 16)]
                        for i in range(16):
                            val = ev16[i]
                            e = g * 16 + i
                            for v in range(HALF // 16):
                                sl = gbuf[e, pl.ds(v * 16, 16)]
                                sbuf[e, pl.ds(v * 16, 16)] = sl * val
                        return u

                    lax.fori_loop(0, K // 16, scale, 0)
                    pltpu.async_copy(sbuf, acc.at[rowv.at[j]], ssems[ph],
                                     add=True)
                return cc

            lax.fori_loop(0, CH // 2, pair_body, 0)
            for ph in range(2):
                pltpu.make_async_copy(sbufs[ph], acc.at[rowv.at[CH - 2 + ph]],
                                      ssems[ph]).wait()
            return carry

        lax.fori_loop(0, NCH, chunk_body, 0)
        plsc.subcore_barrier()
        pltpu.sync_copy(acc.at[pl.ds(s * ROWS_PT, ROWS_PT)],
                        out.at[pl.ds(s * ROWS_PT, ROWS_PT), pl.ds(col0, HALF)])

        @pl.when(s == 0)
        def _():
            pltpu.sync_copy(acc.at[pl.ds(REM_BASE, REM)],
                            out.at[pl.ds(REM_BASE, REM), pl.ds(col0, HALF)])

    @pl.when(c == 0)
    def _():
        _run(xt0, 0)

    @pl.when(c == 1)
    def _():
        _run(xt1, HALF)


def _aggregate(xt0, xt1, col4, row4, ev4):
    mesh = plsc.VectorSubcoreMesh(core_axis_name="c", subcore_axis_name="s")
    return pl.kernel(
        _sc_body,
        out_type=jax.ShapeDtypeStruct((N, D), jnp.float32),
        mesh=mesh,
        scratch_types=[
            pltpu.VMEM_SHARED((N, HALF), jnp.float32),
            pltpu.VMEM((CH, K), jnp.int32),
            pltpu.VMEM((CH, K), jnp.int32),
            pltpu.VMEM((CH, K), jnp.float32),
            pltpu.VMEM((K, HALF), jnp.float32),
            pltpu.VMEM((K, HALF), jnp.float32),
            pltpu.VMEM((K, HALF), jnp.float32),
            pltpu.VMEM((K, HALF), jnp.float32),
            pltpu.SemaphoreType.DMA,
            pltpu.SemaphoreType.DMA,
            pltpu.SemaphoreType.DMA,
            pltpu.SemaphoreType.DMA,
        ],
    )(xt0, xt1, col4, row4, ev4)


def kernel(x, edge_index, edge_values, W, b):
    xt0, xt1 = _linear(x, W, b)
    pad = E_PAD - E
    col = jnp.concatenate(
        [edge_index[1].astype(jnp.int32), jnp.zeros((pad,), jnp.int32)])
    row = jnp.concatenate(
        [edge_index[0].astype(jnp.int32), jnp.zeros((pad,), jnp.int32)])
    ev = jnp.concatenate([edge_values, jnp.zeros((pad,), jnp.float32)])
    col4 = col.reshape(NTILES, NCH, CH, K)
    row4 = row.reshape(NTILES, NCH, CH, K)
    ev4 = ev.reshape(NTILES, NCH, CH, K)
    return _aggregate(xt0, xt1, col4, row4, ev4)


# speedup vs baseline: 3.5487x; 1.0295x over previous
import jax
import jax.numpy as jnp
from jax import lax
from jax.experimental import pallas as pl
from jax.experimental.pallas import tpu as pltpu
from jax.experimental.pallas import tpu_sc as plsc

N = 10000
E = 160000
D = 256
HALF = 128
NTILES = 16
K = 64
NB = 160
NCH = 20
CH = NB // NCH
E_PAD = NTILES * NB * K
ROWS_PT = 624
REM_BASE = NTILES * ROWS_PT
REM = N - REM_BASE


def _linear_kernel(x_ref, w_ref, b_ref, o0_ref, o1_ref):
    y = lax.dot_general(x_ref[...], w_ref[...], (((1,), (1,)), ((), ())),
                        preferred_element_type=jnp.float32)
    y = y + b_ref[...]
    o0_ref[...] = y[:, :HALF]
    o1_ref[...] = y[:, HALF:]


def _linear(x, W, b):
    RB = 1000
    return pl.pallas_call(
        _linear_kernel,
        grid=(N // RB,),
        in_specs=[
            pl.BlockSpec((RB, D), lambda i: (i, 0)),
            pl.BlockSpec((D, D), lambda i: (0, 0)),
            pl.BlockSpec((1, D), lambda i: (0, 0)),
        ],
        out_specs=[
            pl.BlockSpec((RB, HALF), lambda i: (i, 0)),
            pl.BlockSpec((RB, HALF), lambda i: (i, 0)),
        ],
        out_shape=[
            jax.ShapeDtypeStruct((N, HALF), jnp.float32),
            jax.ShapeDtypeStruct((N, HALF), jnp.float32),
        ],
    )(x, W, b.reshape(1, D))


def _sc_body(xt0, xt1, col_r, row_r, ev_r, out,
             acc, colv0, rowv0, evv0, colv1, rowv1, evv1,
             gbuf0, gbuf1, sbuf0, sbuf1,
             gsem0, gsem1, ssem0, ssem1, stsem):
    c = lax.axis_index("c")
    s = lax.axis_index("s")

    zero16 = jnp.zeros((16,), jnp.float32)

    def _zrow(i, carry):
        for v in range(HALF // 16):
            sbuf0[i, pl.ds(v * 16, 16)] = zero16
        return carry

    lax.fori_loop(0, K, _zrow, 0)
    nfull = ROWS_PT // K
    for k in range(nfull):
        pltpu.sync_copy(sbuf0, acc.at[pl.ds(s * ROWS_PT + k * K, K)])
    rem = ROWS_PT - nfull * K
    if rem:
        pltpu.sync_copy(sbuf0.at[pl.ds(0, rem)],
                        acc.at[pl.ds(s * ROWS_PT + nfull * K, rem)])

    @pl.when(s == 0)
    def _():
        pltpu.sync_copy(sbuf0.at[pl.ds(0, REM)], acc.at[pl.ds(REM_BASE, REM)])

    plsc.subcore_barrier()

    gbufs = (gbuf0, gbuf1)
    gsems = (gsem0, gsem1)
    sbufs = (sbuf0, sbuf1)
    ssems = (ssem0, ssem1)
    colvs = (colv0, colv1)
    rowvs = (rowv0, rowv1)
    evvs = (evv0, evv1)
    T = CH // 2

    def _run(xt, col0):
        def gather_start(colv, j, b):
            pltpu.make_async_copy(xt.at[colv.at[j]], gbufs[b], gsems[b]).start()

        def stage_async(ch, dset):
            pltpu.make_async_copy(col_r.at[s].at[ch], colvs[dset], stsem).start()
            pltpu.make_async_copy(row_r.at[s].at[ch], rowvs[dset], stsem).start()
            pltpu.make_async_copy(ev_r.at[s].at[ch], evvs[dset], stsem).start()

        def stage_wait(dset):
            pltpu.make_async_copy(col_r.at[s].at[0], colvs[dset], stsem).wait()
            pltpu.make_async_copy(row_r.at[s].at[0], rowvs[dset], stsem).wait()
            pltpu.make_async_copy(ev_r.at[s].at[0], evvs[dset], stsem).wait()

        pltpu.sync_copy(col_r.at[s].at[0], colvs[0])
        pltpu.sync_copy(row_r.at[s].at[0], rowvs[0])
        pltpu.sync_copy(ev_r.at[s].at[0], evvs[0])
        gather_start(colvs[0], 0, 0)

        def chunk_pair(p, carry):
            for half in range(2):
                cset = half
                nset = 1 - half
                colv, rowv, evv = colvs[cset], rowvs[cset], evvs[cset]

                def pair_body(t, cc, colv=colv, rowv=rowv, evv=evv,
                              half=half, nset=nset):
                    for ph in range(2):
                        j = t * 2 + ph
                        gbuf = gbufs[ph]
                        sbuf = sbufs[ph]

                        if ph == 0:
                            gather_start(colv, j + 1, 1)

                            @pl.when(t == 1)
                            def _():
                                if half == 0:
                                    stage_async(2 * p + 1, nset)
                                else:
                                    @pl.when(p < NCH // 2 - 1)
                                    def _():
                                        stage_async(2 * p + 2, nset)
                        else:
                            @pl.when(t < T - 1)
                            def _():
                                gather_start(colv, j + 1, 0)

                            @pl.when(t == T - 1)
                            def _():
                                if half == 0:
                                    stage_wait(nset)
                                    gather_start(colvs[nset], 0, 0)
                                else:
                                    @pl.when(p < NCH // 2 - 1)
                                    def _():
                                        stage_wait(nset)
                                        gather_start(colvs[nset], 0, 0)

                        if half == 0:
                            @pl.when((t > 0) | (p > 0))
                            def _():
                                pltpu.make_async_copy(
                                    sbuf, acc.at[rowv.at[j]], ssems[ph]).wait()
                        else:
                            pltpu.make_async_copy(
                                sbuf, acc.at[rowv.at[j]], ssems[ph]).wait()

                        pltpu.make_async_copy(xt.at[colv.at[j]], gbuf,
                                              gsems[ph]).wait()

                        def scale(g, u):
                            ev16 = evv[j, pl.ds(g * 16, 16)]
                            for i in range(16):
                                val = ev16[i]
                                e = g * 16 + i
                                for v in range(HALF // 16):
                                    sl = gbuf[e, pl.ds(v * 16, 16)]
                                    sbuf[e, pl.ds(v * 16, 16)] = sl * val
                            return u

                        lax.fori_loop(0, K // 16, scale, 0)
                        pltpu.async_copy(sbuf, acc.at[rowv.at[j]], ssems[ph],
                                         add=True)
                    return cc

                lax.fori_loop(0, T, pair_body, 0)
            return carry

        lax.fori_loop(0, NCH // 2, chunk_pair, 0)

        for ph in range(2):
            pltpu.make_async_copy(sbufs[ph], acc.at[pl.ds(0, K)],
                                  ssems[ph]).wait()

        plsc.subcore_barrier()
        pltpu.sync_copy(acc.at[pl.ds(s * ROWS_PT, ROWS_PT)],
                        out.at[pl.ds(s * ROWS_PT, ROWS_PT), pl.ds(col0, HALF)])

        @pl.when(s == 0)
        def _():
            pltpu.sync_copy(acc.at[pl.ds(REM_BASE, REM)],
                            out.at[pl.ds(REM_BASE, REM), pl.ds(col0, HALF)])

    @pl.when(c == 0)
    def _():
        _run(xt0, 0)

    @pl.when(c == 1)
    def _():
        _run(xt1, HALF)


def _aggregate(xt0, xt1, col4, row4, ev4):
    mesh = plsc.VectorSubcoreMesh(core_axis_name="c", subcore_axis_name="s")
    return pl.kernel(
        _sc_body,
        out_type=jax.ShapeDtypeStruct((N, D), jnp.float32),
        mesh=mesh,
        scratch_types=[
            pltpu.VMEM_SHARED((N, HALF), jnp.float32),
            pltpu.VMEM((CH, K), jnp.int32),
            pltpu.VMEM((CH, K), jnp.int32),
            pltpu.VMEM((CH, K), jnp.float32),
            pltpu.VMEM((CH, K), jnp.int32),
            pltpu.VMEM((CH, K), jnp.int32),
            pltpu.VMEM((CH, K), jnp.float32),
            pltpu.VMEM((K, HALF), jnp.float32),
            pltpu.VMEM((K, HALF), jnp.float32),
            pltpu.VMEM((K, HALF), jnp.float32),
            pltpu.VMEM((K, HALF), jnp.float32),
            pltpu.SemaphoreType.DMA,
            pltpu.SemaphoreType.DMA,
            pltpu.SemaphoreType.DMA,
            pltpu.SemaphoreType.DMA,
            pltpu.SemaphoreType.DMA,
        ],
    )(xt0, xt1, col4, row4, ev4)


def kernel(x, edge_index, edge_values, W, b):
    xt0, xt1 = _linear(x, W, b)
    pad = E_PAD - E
    col = jnp.concatenate(
        [edge_index[1].astype(jnp.int32), jnp.zeros((pad,), jnp.int32)])
    row = jnp.concatenate(
        [edge_index[0].astype(jnp.int32), jnp.zeros((pad,), jnp.int32)])
    ev = jnp.concatenate([edge_values, jnp.zeros((pad,), jnp.float32)])
    col4 = col.reshape(NTILES, NCH, CH, K)
    row4 = row.reshape(NTILES, NCH, CH, K)
    ev4 = ev.reshape(NTILES, NCH, CH, K)
    return _aggregate(xt0, xt1, col4, row4, ev4)
